# baseline (speedup 1.0000x reference)
import functools

import jax
import jax.numpy as jnp
from jax import lax
from jax.experimental import pallas as pl
from jax.experimental.pallas import tpu as pltpu
from jax.experimental.pallas import tpu_sc as plsc

_DEPTH = 5
_R = 2000
_NC = 2
_NS = 16
_NW = _NC * _NS


def _sc_gather(table, idx, ep):
    d = table.shape[1]
    per_w = ep // _NW
    full = per_w // 128
    tail = per_w - full * 128
    mesh = plsc.VectorSubcoreMesh(core_axis_name="c", subcore_axis_name="s")

    @functools.partial(
        pl.kernel,
        out_type=jax.ShapeDtypeStruct((ep, d), jnp.float32),
        mesh=mesh,
        scratch_types=[
            pltpu.VMEM((128,), jnp.int32),
            pltpu.VMEM((128, d), jnp.float32),
        ],
    )
    def k(table_h, idx_h, out_h, idx_v, rows_v):
        wid = lax.axis_index("s") * _NC + lax.axis_index("c")
        base = wid * per_w

        def chunk(off, sz):
            pltpu.sync_copy(idx_h.at[pl.ds(off, sz)], idx_v.at[pl.ds(0, sz)])
            pltpu.sync_copy(
                table_h.at[idx_v.at[pl.ds(0, sz)]], rows_v.at[pl.ds(0, sz)]
            )
            pltpu.sync_copy(rows_v.at[pl.ds(0, sz)], out_h.at[pl.ds(off, sz)])

        @pl.loop(0, full)
        def _(i):
            chunk(base + i * 128, 128)

        if tail:
            chunk(base + full * 128, tail)

    return k(table, idx)


_BLK = 8448


def _sc_scatter(g2, s1, d1, o1, s0, d0, o0, npad, nblk):
    dimh = g2.shape[1]
    rpt = _BLK // _NS
    zr = 66
    grp = 3
    mesh = plsc.VectorSubcoreMesh(core_axis_name="c", subcore_axis_name="s")

    @functools.partial(
        pl.kernel,
        out_type=jax.ShapeDtypeStruct((npad, dimh), jnp.float32),
        mesh=mesh,
        scratch_types=[
            pltpu.VMEM((80,), jnp.int32),
            pltpu.VMEM((80,), jnp.int32),
            pltpu.VMEM((zr, dimh), jnp.float32),
            pltpu.VMEM((grp * 128, dimh), jnp.float32),
            pltpu.VMEM_SHARED((_BLK + 8, dimh), jnp.float32),
            pltpu.SemaphoreType.DMA,
            pltpu.SemaphoreType.DMA,
            pltpu.SemaphoreType.DMA,
        ] + [pltpu.VMEM((128,), jnp.int32) for _ in range(4 * grp)],
        compiler_params=pltpu.CompilerParams(needs_layout_passes=False),
    )
    def k(g_h, s1_h, d1_h, o1_h, s0_h, d0_h, o0_h, m_h,
          off1_v, off0_v, zbuf, rows, acc_sh, sem_g, sem_s, sem_z, *idxbufs):
        sbufs = [idxbufs[p * grp + c] for p in range(2) for c in range(grp)]
        dbufs = [idxbufs[2 * grp + p * grp + c] for p in range(2) for c in range(grp)]
        cidx = lax.axis_index("c")
        tid = lax.axis_index("s")
        pltpu.sync_copy(o1_h, off1_v)
        pltpu.sync_copy(o0_h, off0_v)

        @pl.loop(0, zr)
        def _(r):
            for j in range(dimh // 16):
                zbuf[r, pl.ds(j * 16, 16)] = jnp.zeros((16,), jnp.float32)

        def drain_scatters(count):
            @pl.loop(0, count)
            def _(i):
                pltpu.make_async_copy(
                    rows.at[pl.ds(0, 128)], acc_sh.at[dbufs[0]], sem_s
                ).wait()

        def process_list(b, off_v, s_h, d_h):
            bb = jnp.full((16,), 0, jnp.int32) + b
            start = jnp.max(plsc.load_gather(off_v, [bb]))
            end = jnp.max(plsc.load_gather(off_v, [bb + 1]))
            cnt = end - start
            my_start = start + ((cnt * tid) >> 4)
            my_end = start + ((cnt * (tid + 1)) >> 4)
            abase0 = my_start & -128
            ngrp = jnp.where(
                cnt > 0, lax.div(my_end - abase0 + grp * 128 - 1, grp * 128), 0
            )

            def do_group(g, par, pend):
                gb = abase0 + g * (grp * 128)
                for cc in range(grp):
                    ofs = pl.multiple_of(gb + cc * 128, 8)
                    pltpu.async_copy(
                        s_h.at[pl.ds(ofs, 128)], sbufs[par * grp + cc], sem_z)
                    pltpu.async_copy(
                        d_h.at[pl.ds(ofs, 128)], dbufs[par * grp + cc], sem_z)
                for cc in range(grp):
                    pltpu.make_async_copy(
                        s_h.at[pl.ds(0, 128)], sbufs[0], sem_z).wait()
                    pltpu.make_async_copy(
                        d_h.at[pl.ds(0, 128)], dbufs[0], sem_z).wait()
                for cc in range(grp):
                    sb = sbufs[par * grp + cc]
                    db = dbufs[par * grp + cc]
                    for j in range(8):
                        pos = gb + (cc * 128 + j * 16) + lax.iota(jnp.int32, 16)
                        okm = (pos >= my_start) & (pos < my_end)
                        sb[pl.ds(j * 16, 16)] = jnp.where(
                            okm, sb[pl.ds(j * 16, 16)], 0)
                        db[pl.ds(j * 16, 16)] = jnp.where(
                            okm, db[pl.ds(j * 16, 16)], _BLK)
                drain_scatters(pend)
                nc = jnp.clip(lax.div(my_end - gb + 127, 128), 0, grp)
                for c in range(grp):
                    @pl.when(c < nc)
                    def _():
                        pltpu.async_copy(
                            g_h.at[sbufs[par * grp + c]],
                            rows.at[pl.ds(c * 128, 128)], sem_g,
                        )
                for c in range(grp):
                    @pl.when(c < nc)
                    def _():
                        pltpu.make_async_copy(
                            g_h.at[sbufs[par * grp]], rows.at[pl.ds(0, 128)],
                            sem_g,
                        ).wait()
                for c in range(grp):
                    @pl.when(c < nc)
                    def _():
                        pltpu.async_copy(
                            rows.at[pl.ds(c * 128, 128)],
                            acc_sh.at[dbufs[par * grp + c]], sem_s, add=True,
                        )
                return nc

            @pl.loop(0, lax.div(ngrp + 1, 2), init_carry=jnp.int32(0))
            def pairs(p, pend):
                pend = do_group(2 * p, 0, pend)
                pend = do_group(2 * p + 1, 1, pend)
                return pend

            drain_scatters(pairs if pairs is not None else jnp.int32(0))

        @pl.loop(0, nblk // 2)
        def _(i):
            b = i * 2 + cidx
            for q in range(rpt // zr):
                pltpu.async_copy(
                    zbuf, acc_sh.at[pl.ds(tid * rpt + q * zr, zr)], sem_z
                )
            for q in range(rpt // zr):
                pltpu.make_async_copy(
                    zbuf, acc_sh.at[pl.ds(tid * rpt, zr)], sem_z
                ).wait()
            plsc.subcore_barrier()
            process_list(b, off1_v, s1_h, d1_h)
            process_list(b, off0_v, s0_h, d0_h)
            plsc.subcore_barrier()
            pltpu.sync_copy(
                acc_sh.at[pl.ds(tid * rpt, rpt)],
                m_h.at[pl.ds(b * _BLK + tid * rpt, rpt)],
            )
            plsc.subcore_barrier()

    return k(g2, s1, d1, o1, s0, d0, o0)


def _k1_body(xg_ref, ea_ref, w1_ref, w2_ref, b_ref, o_ref):
    acc = jnp.dot(xg_ref[...], w1_ref[...], preferred_element_type=jnp.float32)
    acc += jnp.dot(ea_ref[...], w2_ref[...], preferred_element_type=jnp.float32)
    o_ref[...] = jnp.maximum(acc + b_ref[...], 0.0)


def _k1_rows(xg, ea, w1, w2, b, e):
    d = xg.shape[1]
    bd = ea.shape[1]
    h = w1.shape[1]
    return pl.pallas_call(
        _k1_body,
        grid=(e // _R,),
        in_specs=[
            pl.BlockSpec((_R, d), lambda i: (i, 0)),
            pl.BlockSpec((_R, bd), lambda i: (i, 0)),
            pl.BlockSpec((d, h), lambda i: (0, 0)),
            pl.BlockSpec((bd, h), lambda i: (0, 0)),
            pl.BlockSpec((1, h), lambda i: (0, 0)),
        ],
        out_specs=pl.BlockSpec((_R, h), lambda i: (i, 0)),
        out_shape=jax.ShapeDtypeStruct((e, h), jnp.float32),
    )(xg, ea, w1, w2, b)


def _k1s_body(xg_ref, ea_ref, w1_ref, w2_ref, b_ref, o_ref):
    acc = jnp.dot(xg_ref[...], w1_ref[...], preferred_element_type=jnp.float32)
    acc += jnp.dot(ea_ref[...], w2_ref[...], preferred_element_type=jnp.float32)
    hv = jnp.maximum(acc + b_ref[...], 0.0)
    o_ref[0] = hv
    o_ref[1] = -hv


def _k1s(xg, ea, w1, w2, b, e):
    d = xg.shape[1]
    bd = ea.shape[1]
    h = w1.shape[1]
    return pl.pallas_call(
        _k1s_body,
        grid=(e // _R,),
        in_specs=[
            pl.BlockSpec((_R, d), lambda i: (i, 0)),
            pl.BlockSpec((_R, bd), lambda i: (i, 0)),
            pl.BlockSpec((d, h), lambda i: (0, 0)),
            pl.BlockSpec((bd, h), lambda i: (0, 0)),
            pl.BlockSpec((1, h), lambda i: (0, 0)),
        ],
        out_specs=pl.BlockSpec((2, _R, h), lambda i: (0, i, 0)),
        out_shape=jax.ShapeDtypeStruct((2, e, h), jnp.float32),
    )(xg, ea, w1, w2, b)


def _k2s_body(m_ref, g0_ref, w_ref, b_ref, o_ref):
    acc = jnp.dot(m_ref[...], w_ref[...], preferred_element_type=jnp.float32)
    hv = jnp.maximum(g0_ref[0] + acc + b_ref[...], 0.0)
    o_ref[0] = hv
    o_ref[1] = -hv


def _k2s(m, g0, w, b, e):
    h = w.shape[1]
    return pl.pallas_call(
        _k2s_body,
        grid=(e // _R,),
        in_specs=[
            pl.BlockSpec((_R, h), lambda i: (i, 0)),
            pl.BlockSpec((1, _R, h), lambda i: (0, i, 0)),
            pl.BlockSpec((h, h), lambda i: (0, 0)),
            pl.BlockSpec((1, h), lambda i: (0, 0)),
        ],
        out_specs=pl.BlockSpec((2, _R, h), lambda i: (0, i, 0)),
        out_shape=jax.ShapeDtypeStruct((2, e, h), jnp.float32),
    )(m, g0, w, b)


def kernel(x, edge_index, edge_attr, rev_edge_index, W_i, b_i, W_h, b_h, W_o, b_o):
    n, d = x.shape
    ei0 = edge_index[0]
    ei1 = edge_index[1]

    b_i2 = b_i.reshape(1, -1)
    b_h2 = b_h.reshape(1, -1)
    b_o2 = b_o.reshape(1, -1)

    e = ei0.shape[0]
    ep = -(-e // (64 * _NW)) * (64 * _NW)
    ei0_pad = jnp.pad(ei0, (0, ep - e))
    xg = _sc_gather(x, ei0_pad, ep)

    nblk = 2 * (-(-n // (2 * _BLK)))
    npad = nblk * _BLK

    eprows = -(-e // 128) + 24
    ep2 = eprows * 128

    def prep(dst, srcv):
        order = jnp.argsort(dst)
        ds_ = dst[order]
        off = jnp.searchsorted(
            ds_, (jnp.arange(nblk + 1) * _BLK).astype(jnp.int32)
        ).astype(jnp.int32)
        off = jnp.pad(off, (0, 80 - (nblk + 1)))
        src_s = jnp.pad(srcv[order].astype(jnp.int32), (0, ep2 - e))
        dl = jnp.pad((ds_ % _BLK).astype(jnp.int32), (0, ep2 - e))
        return src_s, dl, off

    s1, d1, o1 = prep(ei1, jnp.arange(e, dtype=jnp.int32))
    s0, d0, o0 = prep(ei0, rev_edge_index + e)
    o_zero = jnp.zeros((80,), jnp.int32)

    g = _k1s(xg, edge_attr, W_i[:d], W_i[d:], b_i2, e)
    g0 = g
    for _ in range(1, _DEPTH):
        m = _sc_scatter(g.reshape(2 * e, -1), s1, d1, o1, s0, d0, o0, npad, nblk)
        g = _k2s(m, g0, W_h, b_h2, e)
    m_final = _sc_scatter(g.reshape(2 * e, -1), s1, d1, o1, s0, d0, o_zero, npad, nblk)
    out = _k1_rows(x, m_final, W_o[:d], W_o[d:], b_o2, n)
    return out


# speedup vs baseline: 5.1608x; 5.1608x over previous
import functools

import jax
import jax.numpy as jnp
from jax import lax
from jax.experimental import pallas as pl
from jax.experimental.pallas import tpu as pltpu
from jax.experimental.pallas import tpu_sc as plsc

_DEPTH = 5
_R = 2000
_NC = 2
_NS = 16
_NW = _NC * _NS


def _sc_gather(table, idx, ep):
    d = table.shape[1]
    per_w = ep // _NW
    full = per_w // 128
    tail = per_w - full * 128
    mesh = plsc.VectorSubcoreMesh(core_axis_name="c", subcore_axis_name="s")

    @functools.partial(
        pl.kernel,
        out_type=jax.ShapeDtypeStruct((ep, d), jnp.float32),
        mesh=mesh,
        scratch_types=[
            pltpu.VMEM((128,), jnp.int32),
            pltpu.VMEM((128, d), jnp.float32),
        ],
    )
    def k(table_h, idx_h, out_h, idx_v, rows_v):
        wid = lax.axis_index("s") * _NC + lax.axis_index("c")
        base = wid * per_w

        def chunk(off, sz):
            pltpu.sync_copy(idx_h.at[pl.ds(off, sz)], idx_v.at[pl.ds(0, sz)])
            pltpu.sync_copy(
                table_h.at[idx_v.at[pl.ds(0, sz)]], rows_v.at[pl.ds(0, sz)]
            )
            pltpu.sync_copy(rows_v.at[pl.ds(0, sz)], out_h.at[pl.ds(off, sz)])

        @pl.loop(0, full)
        def _(i):
            chunk(base + i * 128, 128)

        if tail:
            chunk(base + full * 128, tail)

    return k(table, idx)


_BLK = 11520


def _sc_scatter(g2, sd1, o1, sd0, o0, npad, nblk):
    dimh = g2.shape[1]
    rpt = _BLK // _NS
    mesh = plsc.VectorSubcoreMesh(core_axis_name="c", subcore_axis_name="s")

    @functools.partial(
        pl.kernel,
        out_type=jax.ShapeDtypeStruct((npad, dimh), jnp.float32),
        mesh=mesh,
        scratch_types=[
            pltpu.VMEM((80,), jnp.int32),
            pltpu.VMEM((80,), jnp.int32),
            pltpu.VMEM((128, dimh), jnp.float32),
            pltpu.VMEM((256,), jnp.int32),
            pltpu.VMEM((128, dimh), jnp.float32),
            pltpu.VMEM_SHARED((_BLK + 8, dimh), jnp.float32),
        ],
        compiler_params=pltpu.CompilerParams(needs_layout_passes=False),
    )
    def k(g_h, sd1_h, o1_h, sd0_h, o0_h, m_h,
          off1_v, off0_v, zbuf, sdbuf, rows, acc_sh):
        cidx = lax.axis_index("c")
        tid = lax.axis_index("s")
        pltpu.sync_copy(o1_h, off1_v)
        pltpu.sync_copy(o0_h, off0_v)

        @pl.loop(0, 128)
        def _(r):
            for j in range(dimh // 16):
                zbuf[r, pl.ds(j * 16, 16)] = jnp.zeros((16,), jnp.float32)

        def scatter_list(b, off_v, sd_h):
            bb = jnp.full((16,), 0, jnp.int32) + b
            start = jnp.max(plsc.load_gather(off_v, [bb]))
            end = jnp.max(plsc.load_gather(off_v, [bb + 1]))
            c0 = start >> 7
            nch = ((end + 127) >> 7) - c0
            myn = (nch - tid + 15) >> 4

            @pl.loop(0, myn)
            def _(kk):
                c = c0 + tid + kk * 16
                pltpu.sync_copy(sd_h.at[pl.ds(c * 256, 256)], sdbuf)
                for j in range(8):
                    pos = c * 128 + j * 16 + lax.iota(jnp.int32, 16)
                    okm = (pos >= start) & (pos < end)
                    sv = sdbuf[pl.ds(j * 16, 16)]
                    dv = sdbuf[pl.ds(128 + j * 16, 16)]
                    sdbuf[pl.ds(j * 16, 16)] = jnp.where(okm, sv, 0)
                    sdbuf[pl.ds(128 + j * 16, 16)] = jnp.where(okm, dv, _BLK)
                pltpu.sync_copy(g_h.at[sdbuf.at[pl.ds(0, 128)]], rows)
                pltpu.sync_copy(
                    rows, acc_sh.at[sdbuf.at[pl.ds(128, 128)]], add=True
                )

        @pl.loop(0, nblk // 2)
        def _(i):
            b = i * 2 + cidx
            for q in range(8):
                pltpu.sync_copy(
                    zbuf.at[pl.ds(0, rpt // 8)],
                    acc_sh.at[pl.ds(tid * rpt + q * (rpt // 8), rpt // 8)],
                )
            plsc.subcore_barrier()
            scatter_list(b, off1_v, sd1_h)
            scatter_list(b, off0_v, sd0_h)
            plsc.subcore_barrier()
            pltpu.sync_copy(
                acc_sh.at[pl.ds(tid * rpt, rpt)],
                m_h.at[pl.ds(b * _BLK + tid * rpt, rpt)],
            )
            plsc.subcore_barrier()

    return k(g2, sd1, o1, sd0, o0)


def _k1_body(xg_ref, ea_ref, w1_ref, w2_ref, b_ref, o_ref):
    acc = jnp.dot(xg_ref[...], w1_ref[...], preferred_element_type=jnp.float32)
    acc += jnp.dot(ea_ref[...], w2_ref[...], preferred_element_type=jnp.float32)
    o_ref[...] = jnp.maximum(acc + b_ref[...], 0.0)


def _k1_rows(xg, ea, w1, w2, b, e):
    d = xg.shape[1]
    bd = ea.shape[1]
    h = w1.shape[1]
    return pl.pallas_call(
        _k1_body,
        grid=(e // _R,),
        in_specs=[
            pl.BlockSpec((_R, d), lambda i: (i, 0)),
            pl.BlockSpec((_R, bd), lambda i: (i, 0)),
            pl.BlockSpec((d, h), lambda i: (0, 0)),
            pl.BlockSpec((bd, h), lambda i: (0, 0)),
            pl.BlockSpec((1, h), lambda i: (0, 0)),
        ],
        out_specs=pl.BlockSpec((_R, h), lambda i: (i, 0)),
        out_shape=jax.ShapeDtypeStruct((e, h), jnp.float32),
    )(xg, ea, w1, w2, b)


def _k1s_body(xg_ref, ea_ref, w1_ref, w2_ref, b_ref, o_ref):
    acc = jnp.dot(xg_ref[...], w1_ref[...], preferred_element_type=jnp.float32)
    acc += jnp.dot(ea_ref[...], w2_ref[...], preferred_element_type=jnp.float32)
    hv = jnp.maximum(acc + b_ref[...], 0.0)
    o_ref[0] = hv
    o_ref[1] = -hv


def _k1s(xg, ea, w1, w2, b, e):
    d = xg.shape[1]
    bd = ea.shape[1]
    h = w1.shape[1]
    return pl.pallas_call(
        _k1s_body,
        grid=(e // _R,),
        in_specs=[
            pl.BlockSpec((_R, d), lambda i: (i, 0)),
            pl.BlockSpec((_R, bd), lambda i: (i, 0)),
            pl.BlockSpec((d, h), lambda i: (0, 0)),
            pl.BlockSpec((bd, h), lambda i: (0, 0)),
            pl.BlockSpec((1, h), lambda i: (0, 0)),
        ],
        out_specs=pl.BlockSpec((2, _R, h), lambda i: (0, i, 0)),
        out_shape=jax.ShapeDtypeStruct((2, e, h), jnp.float32),
    )(xg, ea, w1, w2, b)


def _k2s_body(m_ref, g0_ref, w_ref, b_ref, o_ref):
    acc = jnp.dot(m_ref[...], w_ref[...], preferred_element_type=jnp.float32)
    hv = jnp.maximum(g0_ref[0] + acc + b_ref[...], 0.0)
    o_ref[0] = hv
    o_ref[1] = -hv


def _k2s(m, g0, w, b, e):
    h = w.shape[1]
    return pl.pallas_call(
        _k2s_body,
        grid=(e // _R,),
        in_specs=[
            pl.BlockSpec((_R, h), lambda i: (i, 0)),
            pl.BlockSpec((1, _R, h), lambda i: (0, i, 0)),
            pl.BlockSpec((h, h), lambda i: (0, 0)),
            pl.BlockSpec((1, h), lambda i: (0, 0)),
        ],
        out_specs=pl.BlockSpec((2, _R, h), lambda i: (0, i, 0)),
        out_shape=jax.ShapeDtypeStruct((2, e, h), jnp.float32),
    )(m, g0, w, b)


def kernel(x, edge_index, edge_attr, rev_edge_index, W_i, b_i, W_h, b_h, W_o, b_o):
    n, d = x.shape
    ei0 = edge_index[0]
    ei1 = edge_index[1]

    b_i2 = b_i.reshape(1, -1)
    b_h2 = b_h.reshape(1, -1)
    b_o2 = b_o.reshape(1, -1)

    e = ei0.shape[0]
    ep = -(-e // (64 * _NW)) * (64 * _NW)
    ei0_pad = jnp.pad(ei0, (0, ep - e))
    xg = _sc_gather(x, ei0_pad, ep)

    nblk = 2 * (-(-n // (2 * _BLK)))
    npad = nblk * _BLK

    eprows = -(-e // 128) + 24
    ep2 = eprows * 128

    def prep(dst, srcv):
        order = jnp.argsort(dst)
        ds_ = dst[order]
        off = jnp.searchsorted(
            ds_, (jnp.arange(nblk + 1) * _BLK).astype(jnp.int32)
        ).astype(jnp.int32)
        off = jnp.pad(off, (0, 80 - (nblk + 1)))
        src_s = jnp.pad(srcv[order].astype(jnp.int32), (0, ep2 - e))
        dl = jnp.pad((ds_ % _BLK).astype(jnp.int32), (0, ep2 - e))
        sd = jnp.stack(
            [src_s.reshape(eprows, 128), dl.reshape(eprows, 128)], axis=1
        ).reshape(2 * ep2)
        return sd, off

    sd1, o1 = prep(ei1, jnp.arange(e, dtype=jnp.int32))
    sd0, o0 = prep(ei0, rev_edge_index + e)
    o_zero = jnp.zeros((80,), jnp.int32)

    g = _k1s(xg, edge_attr, W_i[:d], W_i[d:], b_i2, e)
    g0 = g
    for _ in range(1, _DEPTH):
        m = _sc_scatter(g.reshape(2 * e, -1), sd1, o1, sd0, o0, npad, nblk)
        g = _k2s(m, g0, W_h, b_h2, e)
    m_final = _sc_scatter(g.reshape(2 * e, -1), sd1, o1, sd0, o_zero, npad, nblk)
    out = _k1_rows(x, m_final, W_o[:d], W_o[d:], b_o2, n)
    return out
